# Initial kernel scaffold; baseline (speedup 1.0000x reference)
import jax
import jax.numpy as jnp
from jax.experimental import pallas as pl


def kernel(p3, p4, p5, p6, p7, cls_tw, cls_tb, bbox_tw, bbox_tb, wcls, bcls, wbox, bbox_b, wctr, bctr, scales):
    raise NotImplementedError("write your pallas kernel here")


# speedup vs baseline: 1.1564x; 1.1564x over previous
import functools

import jax
import jax.numpy as jnp
import numpy as np
from jax import lax
from jax.experimental import pallas as pl
from jax.experimental.pallas import tpu as pltpu

STRIDES = (8, 16, 32, 64, 128)
NUM_CLASSES = 80
PRE_NMS_THRESH = 0.05
PRE_NMS_TOP_N = 1000
NMS_THRESH = 0.6
MAX_DET = 100
IMG_SIZE = 800.0
SCALE_CLAMP = float(np.log(1000.0 / 16.0))

CH = 256
NCLS_PAD = 128

LEVEL_CFG = ((100, 100, 4), (50, 50, 1), (25, 25, 1), (13, 13, 1), (7, 7, 1))


def _wpad(w):
    return ((w + 8) + 7) // 8 * 8


def _tower_kernel(x_ref, wc_ref, wb_ref, tbc_ref, tbb_ref, hc_ref, hb_ref,
                  hbc_ref, hbb_ref, mask_ref, sc_ref, db_ref, abuf, bbuf,
                  *, M, Wp):
    Mb = M + 2 * Wp
    mask = mask_ref[0]

    def conv_tower(w_ref, tb_ref):
        zrow = jnp.zeros((Wp, CH), jnp.bfloat16)
        abuf[0:Wp] = zrow
        abuf[Wp + M:Mb] = zrow
        abuf[Wp:Wp + M] = x_ref[0, 0]
        bbuf[0:Wp] = zrow
        bbuf[Wp + M:Mb] = zrow
        cur, nxt = abuf, bbuf
        for layer in range(4):
            c0 = cur[...]
            cl = jnp.roll(c0, 1, axis=0)
            cr = jnp.roll(c0, -1, axis=0)
            acc = None
            for ky in range(3):
                base = ky * Wp
                for kx, arr in ((0, cl), (1, c0), (2, cr)):
                    piece = arr[base:base + M]
                    d = lax.dot_general(
                        piece, w_ref[layer, ky, kx],
                        (((1,), (0,)), ((), ())),
                        preferred_element_type=jnp.float32)
                    acc = d if acc is None else acc + d
            act = jnp.maximum(acc + tb_ref[layer:layer + 1, :], 0.0)
            nxt[Wp:Wp + M] = (act * mask).astype(jnp.bfloat16)
            cur, nxt = nxt, cur
        return cur[Wp:Wp + M]

    fc = conv_tower(wc_ref, tbc_ref)
    logits = lax.dot_general(fc, hc_ref[...], (((1,), (0,)), ((), ())),
                             preferred_element_type=jnp.float32) + hbc_ref[...]
    fb = conv_tower(wb_ref, tbb_ref)
    hbo = lax.dot_general(fb, hb_ref[...], (((1,), (0,)), ((), ())),
                          preferred_element_type=jnp.float32) + hbb_ref[...]
    ctr_sig = jax.nn.sigmoid(hbo[:, 4:5])
    s = jax.nn.sigmoid(logits) * ctr_sig
    s = jnp.where(s > PRE_NMS_THRESH, s, 0.0)
    sc_ref[0, 0] = s
    db_ref[0, 0] = hbo[:, 0:8]


def _run_level(xf, wc, wb, tbc, tbb, hc, hb, hbc, hbb, mask, *, N, T, M, Wp):
    Mb = M + 2 * Wp
    kern = functools.partial(_tower_kernel, M=M, Wp=Wp)
    full = lambda shape: pl.BlockSpec(shape, lambda n, t: tuple([0] * len(shape)))
    sc, db = pl.pallas_call(
        kern,
        grid=(N, T),
        in_specs=[
            pl.BlockSpec((1, 1, M, CH), lambda n, t: (n, t, 0, 0)),
            full((4, 3, 3, CH, CH)),
            full((4, 3, 3, CH, CH)),
            full((4, CH)),
            full((4, CH)),
            full((CH, NCLS_PAD)),
            full((CH, NCLS_PAD)),
            full((1, NCLS_PAD)),
            full((1, NCLS_PAD)),
            pl.BlockSpec((1, M, 1), lambda n, t: (t, 0, 0)),
        ],
        out_specs=[
            pl.BlockSpec((1, 1, M, NCLS_PAD), lambda n, t: (n, t, 0, 0)),
            pl.BlockSpec((1, 1, M, 8), lambda n, t: (n, t, 0, 0)),
        ],
        out_shape=[
            jax.ShapeDtypeStruct((N, T, M, NCLS_PAD), jnp.float32),
            jax.ShapeDtypeStruct((N, T, M, 8), jnp.float32),
        ],
        scratch_shapes=[
            pltpu.VMEM((Mb, CH), jnp.bfloat16),
            pltpu.VMEM((Mb, CH), jnp.bfloat16),
        ],
        compiler_params=pltpu.CompilerParams(
            dimension_semantics=("parallel", "arbitrary"),
            vmem_limit_bytes=52 * 1024 * 1024,
        ),
        name="atss_tower",
    )(xf, wc, wb, tbc, tbb, hc, hb, hbc, hbb, mask)
    return sc, db


NPAD = 5120


def _nms_kernel(vals_ref, cls_ref, dts_ref, anc_ref, svec_ref, out_ref):
    v = vals_ref[0]
    c = cls_ref[0]
    sv = svec_ref[0]

    d0 = dts_ref[0, 0] * sv
    d1 = dts_ref[0, 1] * sv
    d2 = dts_ref[0, 2] * sv
    d3 = dts_ref[0, 3] * sv
    a0 = anc_ref[0, 0]
    a1 = anc_ref[0, 1]
    a2 = anc_ref[0, 2]
    a3 = anc_ref[0, 3]

    aw = a2 - a0
    ah = a3 - a1
    acx = a0 + 0.5 * aw
    acy = a1 + 0.5 * ah
    dx = d0 / 10.0
    dy = d1 / 10.0
    dw = jnp.minimum(d2 / 5.0, SCALE_CLAMP)
    dh = jnp.minimum(d3 / 5.0, SCALE_CLAMP)
    pcx = dx * aw + acx
    pcy = dy * ah + acy
    pw = jnp.exp(dw) * aw
    ph = jnp.exp(dh) * ah
    bx1 = jnp.clip(pcx - 0.5 * pw, 0.0, IMG_SIZE)
    by1 = jnp.clip(pcy - 0.5 * ph, 0.0, IMG_SIZE)
    bx2 = jnp.clip(pcx + 0.5 * pw, 0.0, IMG_SIZE)
    by2 = jnp.clip(pcy + 0.5 * ph, 0.0, IMG_SIZE)

    sc = v
    off = c * (2.0 * IMG_SIZE)
    ox1 = bx1 + off
    oy1 = by1 + off
    ox2 = bx2 + off
    oy2 = by2 + off
    areas = (ox2 - ox1) * (oy2 - oy1)

    iota = (lax.broadcasted_iota(jnp.int32, (8, 640), 0) * 640
            + lax.broadcasted_iota(jnp.int32, (8, 640), 1))
    tlane = lax.broadcasted_iota(jnp.int32, (8, NCLS_PAD), 1)
    rowi = lax.broadcasted_iota(jnp.int32, (8, NCLS_PAD), 0)
    out0 = jnp.where(rowi == 5, -1.0, 0.0)

    def pick(onehot, arr):
        return jnp.sum(jnp.where(onehot, arr, 0.0))

    def step(t, carry):
        s, out = carry
        m = jnp.max(s)
        pidx = jnp.min(jnp.where(s == m, iota, jnp.int32(2 ** 30)))
        onehot = iota == pidx
        px1 = pick(onehot, ox1)
        py1 = pick(onehot, oy1)
        px2 = pick(onehot, ox2)
        py2 = pick(onehot, oy2)
        pa = pick(onehot, areas)
        pb1 = pick(onehot, bx1)
        pb2 = pick(onehot, by1)
        pb3 = pick(onehot, bx2)
        pb4 = pick(onehot, by2)
        pc = pick(onehot, c)
        ix1 = jnp.maximum(px1, ox1)
        iy1 = jnp.maximum(py1, oy1)
        ix2 = jnp.minimum(px2, ox2)
        iy2 = jnp.minimum(py2, oy2)
        inter = jnp.clip(ix2 - ix1, 0.0, None) * jnp.clip(iy2 - iy1, 0.0, None)
        iou = inter / (pa + areas - inter + 1e-9)
        s = jnp.where(iou > NMS_THRESH, -1.0, s)
        s = jnp.where(onehot, -1.0, s)
        valid = m > 0.0
        newcol = jnp.stack([pb1, pb2, pb3, pb4, m, pc, 0.0, 0.0]).reshape(8, 1)
        out = jnp.where((tlane == t) & valid, newcol, out)
        return s, out

    _, out = lax.fori_loop(0, MAX_DET, step, (sc, out0))
    out_ref[0] = out


def _run_nms(vals, cls_f, dts, anc, svec, *, N):
    return pl.pallas_call(
        _nms_kernel,
        grid=(N,),
        in_specs=[
            pl.BlockSpec((1, 8, 640), lambda n: (n, 0, 0)),
            pl.BlockSpec((1, 8, 640), lambda n: (n, 0, 0)),
            pl.BlockSpec((1, 4, 8, 640), lambda n: (n, 0, 0, 0)),
            pl.BlockSpec((1, 4, 8, 640), lambda n: (n, 0, 0, 0)),
            pl.BlockSpec((1, 8, 640), lambda n: (0, 0, 0)),
        ],
        out_specs=pl.BlockSpec((1, 8, NCLS_PAD), lambda n: (n, 0, 0)),
        out_shape=jax.ShapeDtypeStruct((N, 8, NCLS_PAD), jnp.float32),
        compiler_params=pltpu.CompilerParams(
            dimension_semantics=("parallel",),
        ),
        name="atss_nms",
    )(vals, cls_f, dts, anc, svec)


def _make_anchors(H, W, stride):
    size = 8.0 * stride
    xs = (jnp.arange(W, dtype=jnp.float32) + 0.5) * stride
    ys = (jnp.arange(H, dtype=jnp.float32) + 0.5) * stride
    cy, cx = jnp.meshgrid(ys, xs, indexing='ij')
    half = size / 2.0
    return jnp.stack([cx - half, cy - half, cx + half, cy + half], -1).reshape(-1, 4)


def _level_mask(H, W, T, Wp):
    R = H // T
    rows = R + 8
    M = rows * Wp
    m = np.zeros((T, rows, Wp), np.float32)
    for t in range(T):
        for r in range(rows):
            img_row = R * t + r - 4
            if 0 <= img_row < H:
                m[t, r, 4:4 + W] = 1.0
    return jnp.asarray(m.reshape(T, M, 1))


def kernel(p3, p4, p5, p6, p7, cls_tw, cls_tb, bbox_tw, bbox_tb,
           wcls, bcls, wbox, bbox_b, wctr, bctr, scales):
    feats = [p3, p4, p5, p6, p7]
    N = p3.shape[0]

    wc = cls_tw.transpose(0, 3, 4, 2, 1).astype(jnp.bfloat16)
    wb = bbox_tw.transpose(0, 3, 4, 2, 1).astype(jnp.bfloat16)
    hc = jnp.pad(wcls.T, ((0, 0), (0, NCLS_PAD - NUM_CLASSES))).astype(jnp.bfloat16)
    hb = jnp.pad(jnp.concatenate([wbox.T, wctr.T], axis=1),
                 ((0, 0), (0, NCLS_PAD - 5))).astype(jnp.bfloat16)
    hbc = jnp.pad(bcls, (0, NCLS_PAD - NUM_CLASSES)).reshape(1, NCLS_PAD)
    hbb = jnp.pad(jnp.concatenate([bbox_b, bctr]), (0, NCLS_PAD - 5)).reshape(1, NCLS_PAD)

    all_vals, all_cls, all_d, all_anc = [], [], [], []
    for l, x in enumerate(feats):
        H, W, T = LEVEL_CFG[l]
        bfc = jnp.bfloat16
        c = x
        b = x
        for i in range(4):
            c = jax.nn.relu(lax.conv_general_dilated(
                c.astype(bfc), cls_tw[i].astype(bfc), (1, 1), 'SAME',
                dimension_numbers=('NCHW', 'OIHW', 'NCHW'),
                preferred_element_type=jnp.float32) + cls_tb[i][None, :, None, None])
            b = jax.nn.relu(lax.conv_general_dilated(
                b.astype(bfc), bbox_tw[i].astype(bfc), (1, 1), 'SAME',
                dimension_numbers=('NCHW', 'OIHW', 'NCHW'),
                preferred_element_type=jnp.float32) + bbox_tb[i][None, :, None, None])
        logits = jnp.einsum('nchw,kc->nkhw', c.astype(bfc), wcls.astype(bfc),
                            preferred_element_type=jnp.float32) + bcls[None, :, None, None]
        deltas_r = jnp.einsum('nchw,kc->nkhw', b.astype(bfc), wbox.astype(bfc),
                              preferred_element_type=jnp.float32) + bbox_b[None, :, None, None]
        ctr = jnp.einsum('nchw,kc->nkhw', b.astype(bfc), wctr.astype(bfc),
                         preferred_element_type=jnp.float32) + bctr[None, :, None, None]
        s = jax.nn.sigmoid(logits) * jax.nn.sigmoid(ctr)
        s = jnp.where(s > PRE_NMS_THRESH, s, 0.0)
        sc = s.transpose(0, 2, 3, 1).reshape(N, H * W, NUM_CLASSES)
        db = deltas_r.transpose(0, 2, 3, 1).reshape(N, H * W, 4)
        flat = sc.reshape(N, H * W * NUM_CLASSES)
        vals, idx = lax.top_k(flat, PRE_NMS_TOP_N)
        aidx = idx // NUM_CLASSES
        cidx = idx % NUM_CLASSES
        d = jnp.take_along_axis(db, aidx[:, :, None], axis=1)
        anc = _make_anchors(H, W, STRIDES[l])[aidx]
        sc_fin = jnp.sqrt(jnp.where(vals > 0, vals, 1.0)) * (vals > 0)
        all_vals.append(sc_fin)
        all_cls.append(cidx)
        all_d.append(d)
        all_anc.append(anc)

    nc = 5 * PRE_NMS_TOP_N
    pad = NPAD - nc
    vals = jnp.pad(jnp.concatenate(all_vals, 1), ((0, 0), (0, pad)))
    cls_f = jnp.pad(jnp.concatenate(all_cls, 1).astype(jnp.float32),
                    ((0, 0), (0, pad)))
    dts = jnp.pad(jnp.concatenate(all_d, 1), ((0, 0), (0, pad), (0, 0)))
    anc = jnp.pad(jnp.concatenate(all_anc, 1), ((0, 0), (0, pad), (0, 0)))
    svec = jnp.pad(jnp.repeat(scales, PRE_NMS_TOP_N), (0, pad))

    vals = vals.reshape(N, 8, 640)
    cls_f = cls_f.reshape(N, 8, 640)
    dts = dts.transpose(0, 2, 1).reshape(N, 4, 8, 640)
    anc = anc.transpose(0, 2, 1).reshape(N, 4, 8, 640)
    svec = svec.reshape(1, 8, 640)

    out = _run_nms(vals, cls_f, dts, anc, svec, N=N)
    boxes = out[:, :4, :MAX_DET].transpose(0, 2, 1)
    scores = out[:, 4, :MAX_DET]
    classes = out[:, 5, :MAX_DET].astype(jnp.int32)
    return boxes, scores, classes


# speedup vs baseline: 1.1565x; 1.0001x over previous
import jax
import jax.numpy as jnp
import numpy as np
from jax import lax
from jax.experimental import pallas as pl
from jax.experimental.pallas import tpu as pltpu

STRIDES = (8, 16, 32, 64, 128)
NUM_CLASSES = 80
PRE_NMS_THRESH = 0.05
PRE_NMS_TOP_N = 1000
NMS_THRESH = 0.6
MAX_DET = 100
IMG_SIZE = 800.0
SCALE_CLAMP = float(np.log(1000.0 / 16.0))

NCLS_PAD = 128

LEVEL_CFG = ((100, 100), (50, 50), (25, 25), (13, 13), (7, 7))


NPAD = 5120


def _nms_kernel(vals_ref, cls_ref, dts_ref, anc_ref, svec_ref, out_ref):
    v = vals_ref[0]
    c = cls_ref[0]
    sv = svec_ref[0]

    d0 = dts_ref[0, 0] * sv
    d1 = dts_ref[0, 1] * sv
    d2 = dts_ref[0, 2] * sv
    d3 = dts_ref[0, 3] * sv
    a0 = anc_ref[0, 0]
    a1 = anc_ref[0, 1]
    a2 = anc_ref[0, 2]
    a3 = anc_ref[0, 3]

    aw = a2 - a0
    ah = a3 - a1
    acx = a0 + 0.5 * aw
    acy = a1 + 0.5 * ah
    dx = d0 / 10.0
    dy = d1 / 10.0
    dw = jnp.minimum(d2 / 5.0, SCALE_CLAMP)
    dh = jnp.minimum(d3 / 5.0, SCALE_CLAMP)
    pcx = dx * aw + acx
    pcy = dy * ah + acy
    pw = jnp.exp(dw) * aw
    ph = jnp.exp(dh) * ah
    bx1 = jnp.clip(pcx - 0.5 * pw, 0.0, IMG_SIZE)
    by1 = jnp.clip(pcy - 0.5 * ph, 0.0, IMG_SIZE)
    bx2 = jnp.clip(pcx + 0.5 * pw, 0.0, IMG_SIZE)
    by2 = jnp.clip(pcy + 0.5 * ph, 0.0, IMG_SIZE)

    sc = v
    off = c * (2.0 * IMG_SIZE)
    ox1 = bx1 + off
    oy1 = by1 + off
    ox2 = bx2 + off
    oy2 = by2 + off
    areas = (ox2 - ox1) * (oy2 - oy1)

    iota = (lax.broadcasted_iota(jnp.int32, (8, 640), 0) * 640
            + lax.broadcasted_iota(jnp.int32, (8, 640), 1))
    tlane = lax.broadcasted_iota(jnp.int32, (8, NCLS_PAD), 1)
    rowi = lax.broadcasted_iota(jnp.int32, (8, NCLS_PAD), 0)
    out0 = jnp.where(rowi == 5, -1.0, 0.0)

    def pick(onehot, arr):
        return jnp.sum(jnp.where(onehot, arr, 0.0))

    def step(t, carry):
        s, out = carry
        m = jnp.max(s)
        pidx = jnp.min(jnp.where(s == m, iota, jnp.int32(2 ** 30)))
        onehot = iota == pidx
        px1 = pick(onehot, ox1)
        py1 = pick(onehot, oy1)
        px2 = pick(onehot, ox2)
        py2 = pick(onehot, oy2)
        pa = pick(onehot, areas)
        pb1 = pick(onehot, bx1)
        pb2 = pick(onehot, by1)
        pb3 = pick(onehot, bx2)
        pb4 = pick(onehot, by2)
        pc = pick(onehot, c)
        ix1 = jnp.maximum(px1, ox1)
        iy1 = jnp.maximum(py1, oy1)
        ix2 = jnp.minimum(px2, ox2)
        iy2 = jnp.minimum(py2, oy2)
        inter = jnp.clip(ix2 - ix1, 0.0, None) * jnp.clip(iy2 - iy1, 0.0, None)
        iou = inter / (pa + areas - inter + 1e-9)
        s = jnp.where(iou > NMS_THRESH, -1.0, s)
        s = jnp.where(onehot, -1.0, s)
        valid = m > 0.0
        newcol = jnp.stack([pb1, pb2, pb3, pb4, m, pc, 0.0, 0.0]).reshape(8, 1)
        out = jnp.where((tlane == t) & valid, newcol, out)
        return s, out

    _, out = lax.fori_loop(0, MAX_DET, step, (sc, out0))
    out_ref[0] = out


def _run_nms(vals, cls_f, dts, anc, svec, *, N):
    return pl.pallas_call(
        _nms_kernel,
        grid=(N,),
        in_specs=[
            pl.BlockSpec((1, 8, 640), lambda n: (n, 0, 0)),
            pl.BlockSpec((1, 8, 640), lambda n: (n, 0, 0)),
            pl.BlockSpec((1, 4, 8, 640), lambda n: (n, 0, 0, 0)),
            pl.BlockSpec((1, 4, 8, 640), lambda n: (n, 0, 0, 0)),
            pl.BlockSpec((1, 8, 640), lambda n: (0, 0, 0)),
        ],
        out_specs=pl.BlockSpec((1, 8, NCLS_PAD), lambda n: (n, 0, 0)),
        out_shape=jax.ShapeDtypeStruct((N, 8, NCLS_PAD), jnp.float32),
        compiler_params=pltpu.CompilerParams(
            dimension_semantics=("parallel",),
        ),
        name="atss_nms",
    )(vals, cls_f, dts, anc, svec)


def _make_anchors(H, W, stride):
    size = 8.0 * stride
    xs = (jnp.arange(W, dtype=jnp.float32) + 0.5) * stride
    ys = (jnp.arange(H, dtype=jnp.float32) + 0.5) * stride
    cy, cx = jnp.meshgrid(ys, xs, indexing='ij')
    half = size / 2.0
    return jnp.stack([cx - half, cy - half, cx + half, cy + half], -1).reshape(-1, 4)


def kernel(p3, p4, p5, p6, p7, cls_tw, cls_tb, bbox_tw, bbox_tb,
           wcls, bcls, wbox, bbox_b, wctr, bctr, scales):
    feats = [p3, p4, p5, p6, p7]
    N = p3.shape[0]

    all_vals, all_cls, all_d, all_anc = [], [], [], []
    for l, x in enumerate(feats):
        H, W = LEVEL_CFG[l]
        bfc = jnp.bfloat16
        c = x
        b = x
        for i in range(4):
            c = jax.nn.relu(lax.conv_general_dilated(
                c.astype(bfc), cls_tw[i].astype(bfc), (1, 1), 'SAME',
                dimension_numbers=('NCHW', 'OIHW', 'NCHW'),
                preferred_element_type=jnp.float32) + cls_tb[i][None, :, None, None])
            b = jax.nn.relu(lax.conv_general_dilated(
                b.astype(bfc), bbox_tw[i].astype(bfc), (1, 1), 'SAME',
                dimension_numbers=('NCHW', 'OIHW', 'NCHW'),
                preferred_element_type=jnp.float32) + bbox_tb[i][None, :, None, None])
        logits = jnp.einsum('nchw,kc->nkhw', c.astype(bfc), wcls.astype(bfc),
                            preferred_element_type=jnp.float32) + bcls[None, :, None, None]
        deltas_r = jnp.einsum('nchw,kc->nkhw', b.astype(bfc), wbox.astype(bfc),
                              preferred_element_type=jnp.float32) + bbox_b[None, :, None, None]
        ctr = jnp.einsum('nchw,kc->nkhw', b.astype(bfc), wctr.astype(bfc),
                         preferred_element_type=jnp.float32) + bctr[None, :, None, None]
        s = jax.nn.sigmoid(logits) * jax.nn.sigmoid(ctr)
        s = jnp.where(s > PRE_NMS_THRESH, s, 0.0)
        sc = s.transpose(0, 2, 3, 1).reshape(N, H * W, NUM_CLASSES)
        db = deltas_r.transpose(0, 2, 3, 1).reshape(N, H * W, 4)
        flat = sc.reshape(N, H * W * NUM_CLASSES)
        vals, idx = lax.top_k(flat, PRE_NMS_TOP_N)
        aidx = idx // NUM_CLASSES
        cidx = idx % NUM_CLASSES
        d = jnp.take_along_axis(db, aidx[:, :, None], axis=1)
        anc = _make_anchors(H, W, STRIDES[l])[aidx]
        sc_fin = jnp.sqrt(jnp.where(vals > 0, vals, 1.0)) * (vals > 0)
        all_vals.append(sc_fin)
        all_cls.append(cidx)
        all_d.append(d)
        all_anc.append(anc)

    nc = 5 * PRE_NMS_TOP_N
    pad = NPAD - nc
    vals = jnp.pad(jnp.concatenate(all_vals, 1), ((0, 0), (0, pad)))
    cls_f = jnp.pad(jnp.concatenate(all_cls, 1).astype(jnp.float32),
                    ((0, 0), (0, pad)))
    dts = jnp.pad(jnp.concatenate(all_d, 1), ((0, 0), (0, pad), (0, 0)))
    anc = jnp.pad(jnp.concatenate(all_anc, 1), ((0, 0), (0, pad), (0, 0)))
    svec = jnp.pad(jnp.repeat(scales, PRE_NMS_TOP_N), (0, pad))

    vals = vals.reshape(N, 8, 640)
    cls_f = cls_f.reshape(N, 8, 640)
    dts = dts.transpose(0, 2, 1).reshape(N, 4, 8, 640)
    anc = anc.transpose(0, 2, 1).reshape(N, 4, 8, 640)
    svec = svec.reshape(1, 8, 640)

    out = _run_nms(vals, cls_f, dts, anc, svec, N=N)
    boxes = out[:, :4, :MAX_DET].transpose(0, 2, 1)
    scores = out[:, 4, :MAX_DET]
    classes = out[:, 5, :MAX_DET].astype(jnp.int32)
    return boxes, scores, classes
